# baseline (speedup 1.0000x reference)
import functools

import jax
import jax.numpy as jnp
from jax import lax
from jax.experimental import pallas as pl
from jax.experimental.pallas import tpu as pltpu
from jax.experimental.pallas import tpu_sc as plsc

B = 16384
D = 32
NC = 2
NS = 16
NW = NC * NS
BPW = B // NW
IDX_MINOR = 128
IDX_ROWS = BPW // IDX_MINOR
CHUNK = 256
NCHUNK = BPW // CHUNK


def _body(uid_hbm, mid_hbm, ut_hbm, mt_hbm, fc_hbm, out_hbm,
          uidraw_v, midraw_v, uidx_v, midx_v, urows_v, mrows_v,
          out_v, fc_v, sem):
    wid = lax.axis_index("s") * NC + lax.axis_index("c")
    base = wid * BPW

    pltpu.sync_copy(uid_hbm.at[pl.ds(base, BPW)], uidraw_v)
    pltpu.sync_copy(mid_hbm.at[pl.ds(base, BPW)], midraw_v)
    pltpu.sync_copy(fc_hbm, fc_v)

    for j in range(IDX_ROWS):
        for k in range(IDX_MINOR // 16):
            off = j * IDX_MINOR + k * 16
            uidx_v[j, pl.ds(k * 16, 16)] = (
                lax.shift_right_logical(uidraw_v[pl.ds(off, 16)], 2))
            midx_v[j, pl.ds(k * 16, 16)] = (
                lax.shift_right_logical(midraw_v[pl.ds(off, 16)], 2))

    w_vec = fc_v[pl.ds(0, 16)]
    b_vec = fc_v[pl.ds(16, 16)]
    lane = lax.iota(jnp.int32, 16)

    for c in range(NCHUNK):
        copies = []
        for j in range(CHUNK // IDX_MINOR):
            jj = c * (CHUNK // IDX_MINOR) + j
            copies.append(pltpu.async_copy(
                ut_hbm.at[uidx_v.at[jj]],
                urows_v.at[pl.ds(j * IDX_MINOR, IDX_MINOR)], sem))
            copies.append(pltpu.async_copy(
                mt_hbm.at[midx_v.at[jj]],
                mrows_v.at[pl.ds(j * IDX_MINOR, IDX_MINOR)], sem))
        for cp in copies:
            cp.wait()

        def group(g, carry):
            res = jnp.zeros((16,), jnp.float32)
            uoffs = (uidraw_v[pl.ds(c * CHUNK + g * 16, 16)] & 3) * D
            moffs = (midraw_v[pl.ds(c * CHUNK + g * 16, 16)] & 3) * D
            for r in range(16):
                i = g * 16 + r
                uoff = uoffs[r]
                moff = moffs[r]
                t = (urows_v[i, pl.ds(uoff, 16)] *
                     mrows_v[i, pl.ds(moff, 16)] +
                     urows_v[i, pl.ds(uoff + 16, 16)] *
                     mrows_v[i, pl.ds(moff + 16, 16)])
                res = jnp.where(lane == r, jnp.sum(t), res)
            y = res * w_vec + b_vec
            out_v[pl.ds(c * CHUNK + g * 16, 16)] = 1.0 / (1.0 + jnp.exp(-y))
            return carry

        lax.fori_loop(0, CHUNK // 16, group, 0)

    pltpu.sync_copy(out_v, out_hbm.at[pl.ds(base, BPW)])


@jax.jit
def _run(uid, mid, ut128, mt128, fc128):
    mesh = plsc.VectorSubcoreMesh(core_axis_name="c", subcore_axis_name="s")
    f = functools.partial(
        pl.kernel,
        mesh=mesh,
        compiler_params=pltpu.CompilerParams(needs_layout_passes=False,
                                             use_tc_tiling_on_sc=True),
        out_type=jax.ShapeDtypeStruct((B,), jnp.float32),
        scratch_types=[
            pltpu.VMEM((BPW,), jnp.int32),
            pltpu.VMEM((BPW,), jnp.int32),
            pltpu.VMEM((IDX_ROWS, IDX_MINOR), jnp.int32),
            pltpu.VMEM((IDX_ROWS, IDX_MINOR), jnp.int32),
            pltpu.VMEM((CHUNK, 128), jnp.float32),
            pltpu.VMEM((CHUNK, 128), jnp.float32),
            pltpu.VMEM((BPW,), jnp.float32),
            pltpu.VMEM((128,), jnp.float32),
            pltpu.SemaphoreType.DMA,
        ],
    )(_body)
    return f(uid, mid, ut128, mt128, fc128)


def kernel(user_ids, movie_ids, u_table, m_table, fc_w, fc_b):
    uid = user_ids.astype(jnp.int32)
    mid = movie_ids.astype(jnp.int32)
    ut128 = u_table.reshape(-1, 128)
    mt128 = m_table.reshape(-1, 128)
    fc128 = jnp.concatenate([
        jnp.full((16,), fc_w.reshape(()), jnp.float32),
        jnp.full((16,), fc_b.reshape(()), jnp.float32),
        jnp.zeros((96,), jnp.float32),
    ])
    out = _run(uid, mid, ut128, mt128, fc128)
    return out.reshape(B, 1)
